# baseline (speedup 1.0000x reference)
import functools

import jax
import jax.numpy as jnp
from jax import lax
from jax.experimental import pallas as pl
from jax.experimental.pallas import tpu as pltpu
from jax.experimental.pallas import tpu_sc as plsc

_NUM_USERS = 100000
_LATENT_DIM = 64
_BATCH = 16384
_ITEM_IDX_OFFSET = _NUM_USERS

_NC = 2
_NS = 16
_L = 16
_NW = _NC * _NS
_B_PER_W = _BATCH // _NW
_CH = 128
_NCH = _B_PER_W // _CH

_mesh = plsc.VectorSubcoreMesh(core_axis_name="c", subcore_axis_name="s")


@functools.partial(
    pl.kernel,
    mesh=_mesh,
    out_type=[
        jax.ShapeDtypeStruct((_BATCH, _LATENT_DIM), jnp.float32),
        jax.ShapeDtypeStruct((_BATCH, _LATENT_DIM), jnp.float32),
    ],
    scratch_types=[
        pltpu.VMEM((_NCH, _CH), jnp.int32),
        pltpu.VMEM((_NCH, _CH), jnp.int32),
        pltpu.VMEM((_NCH, _CH, _LATENT_DIM), jnp.float32),
        pltpu.VMEM((_NCH, _CH, _LATENT_DIM), jnp.float32),
        pltpu.SemaphoreType.DMA,
    ],
    compiler_params=pltpu.CompilerParams(use_tc_tiling_on_sc=False),
)
def _emb_lookup(uids_hbm, iids_hbm, table_hbm, out_u, out_i,
                idx_u, idx_i, rows_u, rows_i, sem):
    wid = lax.axis_index("s") * _NC + lax.axis_index("c")
    base = wid * _B_PER_W

    pltpu.sync_copy(uids_hbm.at[wid], idx_u)
    pltpu.sync_copy(iids_hbm.at[wid], idx_i)

    for j in range(_NCH):
        for k in range(_CH // _L):
            sl = (j, pl.ds(k * _L, _L))
            idx_i[sl] = idx_i[sl] + _ITEM_IDX_OFFSET

    copies = []
    for j in range(_NCH):
        copies.append(
            pltpu.async_copy(table_hbm.at[idx_u.at[j]], rows_u.at[j], sem))
        copies.append(
            pltpu.async_copy(table_hbm.at[idx_i.at[j]], rows_i.at[j], sem))
    for c in copies:
        c.wait()

    for j in range(_NCH):
        pltpu.sync_copy(rows_u.at[j], out_u.at[pl.ds(base + j * _CH, _CH)])
        pltpu.sync_copy(rows_i.at[j], out_i.at[pl.ds(base + j * _CH, _CH)])


def kernel(uids, iids, embedding):
    u3 = uids.astype(jnp.int32).reshape(_NW, _NCH, _CH)
    i3 = iids.astype(jnp.int32).reshape(_NW, _NCH, _CH)
    user_embs, item_embs = _emb_lookup(u3, i3, embedding)
    return (user_embs, item_embs)


# speedup vs baseline: 1.9920x; 1.9920x over previous
import functools

import jax
import jax.numpy as jnp
from jax import lax
from jax.experimental import pallas as pl
from jax.experimental.pallas import tpu as pltpu
from jax.experimental.pallas import tpu_sc as plsc

_NUM_USERS = 100000
_NUM_ROWS = 1100000
_LATENT_DIM = 64
_BATCH = 16384
_ITEM_IDX_OFFSET = _NUM_USERS

_NC = 2
_NS = 16
_NW = _NC * _NS
_NT = 2 * _BATCH
_TPW = _NT // _NW
_CW = 1024
_TAIL_START = (_NUM_ROWS // 128) * 128
_TAIL_ROWS = _NUM_ROWS - _TAIL_START

_mesh = plsc.VectorSubcoreMesh(core_axis_name="c", subcore_axis_name="s")


@functools.partial(
    pl.kernel,
    mesh=_mesh,
    out_type=jax.ShapeDtypeStruct((_NT, 128), jnp.float32),
    scratch_types=[
        pltpu.VMEM((_TPW,), jnp.int32),
        pltpu.VMEM((_TPW,), jnp.int32),
        pltpu.VMEM((_CW // 2, 128), jnp.float32),
        pltpu.VMEM((4, 16, 128), jnp.float32),
        pltpu.VMEM((_TAIL_ROWS // 2, 128), jnp.float32),
        pltpu.SemaphoreType.DMA,
        pltpu.SemaphoreType.DMA,
    ],
    compiler_params=pltpu.CompilerParams(
        use_tc_tiling_on_sc=True, needs_layout_passes=False),
)
def _gather_sorted(sidx_hbm, ord_hbm, tblt_hbm, tail_hbm, g_out,
                   sidx_v, ord_v, chunk, srcbuf, tail_v, sem_ld, sem_sc):
    wid = lax.axis_index("s") * _NC + lax.axis_index("c")
    t0 = wid * _TPW

    pltpu.sync_copy(sidx_hbm.at[pl.ds(t0, _TPW)], sidx_v)
    pltpu.sync_copy(ord_hbm.at[pl.ds(t0, _TPW)], ord_v)
    pltpu.sync_copy(tail_hbm, tail_v)

    lanes = lax.iota(jnp.int32, 16)
    zeros16 = jnp.zeros((16,), jnp.int32)

    def extract(cur):
        c = jnp.minimum(cur, _TPW - 1)
        st = (c >> 4) << 4
        m = lanes == (c & 15)
        iv = jnp.sum(jnp.where(m, sidx_v[pl.ds(st, 16)], 0))
        jv = jnp.sum(jnp.where(m, ord_v[pl.ds(st, 16)], 0))
        return iv, jv

    def finish(cur, jvec):
        batch = cur >> 4

        @pl.when((cur & 15) == 15)
        def _():
            @pl.when(batch >= 3)
            def _():
                pltpu.make_async_copy(
                    g_out.at[pl.ds(0, 16), :], srcbuf.at[0], sem_sc).wait()
            pltpu.async_copy(srcbuf.at[batch & 3], g_out.at[jvec], sem_sc)

    def main_phase(carry0):
        def outer_cond(c):
            cur, iv, _, _ = c
            return (cur < _TPW) & (iv < _TAIL_START)

        def outer_body(c):
            cur, iv, jv, jvec = c
            base = pl.multiple_of(
                jnp.minimum((iv >> 7) << 7, _TAIL_START - _CW), 128)
            cps = [
                pltpu.async_copy(
                    tblt_hbm.at[:, pl.ds(pl.multiple_of(base + 128 * a, 128),
                                         128)],
                    chunk.at[pl.ds(64 * a, 64), :],
                    sem_ld,
                )
                for a in range(_CW // 128)
            ]
            for cp in cps:
                cp.wait()

            def inner_cond(c2):
                cur2, iv2, _, _ = c2
                return (cur2 < _TPW) & (iv2 < base + _CW)

            def inner_body(c2):
                cur2, iv2, jv2, jvec2 = c2
                col = iv2 - base
                a = col >> 7
                within = (col & 127) + zeros16
                slot = cur2 & 15
                for grp in range(4):
                    rows = a * 64 + grp * 16 + lanes
                    vals = plsc.load_gather(chunk, [rows, within])
                    srcbuf[(cur2 >> 4) & 3, slot, pl.ds(grp * 16, 16)] = vals
                jvec3 = jnp.where(lanes == slot, jv2, jvec2)
                finish(cur2, jvec3)
                niv, njv = extract(cur2 + 1)
                return (cur2 + 1, niv, njv, jvec3)

            return lax.while_loop(inner_cond, inner_body, (cur, iv, jv, jvec))

        return lax.while_loop(outer_cond, outer_body, carry0)

    def tail_phase(carry0):
        def cond(c):
            cur, _, _, _ = c
            return cur < _TPW

        def body(c):
            cur, iv, jv, jvec = c
            r = iv - _TAIL_START
            prow = (r >> 1) + zeros16
            cb = (r & 1) << 6
            slot = cur & 15
            for grp in range(4):
                cols = cb + grp * 16 + lanes
                vals = plsc.load_gather(tail_v, [prow, cols])
                srcbuf[(cur >> 4) & 3, slot, pl.ds(grp * 16, 16)] = vals
            jvec2 = jnp.where(lanes == slot, jv, jvec)
            finish(cur, jvec2)
            niv, njv = extract(cur + 1)
            return (cur + 1, niv, njv, jvec2)

        return lax.while_loop(cond, body, carry0)

    iv0, jv0 = extract(0)
    carry = (jnp.int32(0), iv0, jv0, zeros16)
    carry = main_phase(carry)
    tail_phase(carry)

    for _ in range(3):
        pltpu.make_async_copy(
            g_out.at[pl.ds(0, 16), :], srcbuf.at[0], sem_sc).wait()


def kernel(uids, iids, embedding):
    ui = uids.astype(jnp.int32)
    ii = iids.astype(jnp.int32) + _ITEM_IDX_OFFSET
    allidx = jnp.concatenate([ui, ii])
    sidx, order = lax.sort_key_val(
        allidx, jnp.arange(_NT, dtype=jnp.int32))
    tail2 = embedding[_TAIL_START:, :].reshape(_TAIL_ROWS // 2, 128)
    g = _gather_sorted(sidx, order, embedding.T, tail2)
    return (g[:_BATCH, :_LATENT_DIM], g[_BATCH:, :_LATENT_DIM])


# speedup vs baseline: 2.6250x; 1.3177x over previous
import functools

import jax
import jax.numpy as jnp
from jax import lax
from jax.experimental import pallas as pl
from jax.experimental.pallas import tpu as pltpu
from jax.experimental.pallas import tpu_sc as plsc

_NUM_USERS = 100000
_NUM_ROWS = 1100000
_LATENT_DIM = 64
_BATCH = 16384
_ITEM_IDX_OFFSET = _NUM_USERS

_NC = 2
_NS = 16
_NW = _NC * _NS
_NT = 2 * _BATCH
_TPW = _NT // _NW
_NG = _TPW // 16
_CW = 768
_NB = _CW // 128
_TAIL_START = (_NUM_ROWS // 128) * 128
_TAIL_ROWS = _NUM_ROWS - _TAIL_START
_BIG = 2**31 - 1

_mesh = plsc.VectorSubcoreMesh(core_axis_name="c", subcore_axis_name="s")


@functools.partial(
    pl.kernel,
    mesh=_mesh,
    out_type=jax.ShapeDtypeStruct((_NT, 128), jnp.float32),
    scratch_types=[
        pltpu.VMEM((_TPW,), jnp.int32),
        pltpu.VMEM((_TPW,), jnp.int32),
        pltpu.VMEM((2, _CW // 2, 128), jnp.float32),
        pltpu.VMEM((4, 16, 128), jnp.float32),
        pltpu.VMEM((_TAIL_ROWS // 2, 128), jnp.float32),
        pltpu.SemaphoreType.DMA,
        pltpu.SemaphoreType.DMA,
    ],
    compiler_params=pltpu.CompilerParams(
        use_tc_tiling_on_sc=True, needs_layout_passes=False),
)
def _gather_sorted(sidx_hbm, ord_hbm, tblt_hbm, tail_hbm, g_out,
                   sidx_v, ord_v, chunk, srcbuf, tail_v, sem_ld, sem_sc):
    wid = lax.axis_index("s") * _NC + lax.axis_index("c")
    t0 = wid * _TPW

    pltpu.sync_copy(sidx_hbm.at[pl.ds(t0, _TPW)], sidx_v)
    pltpu.sync_copy(ord_hbm.at[pl.ds(t0, _TPW)], ord_v)
    pltpu.sync_copy(tail_hbm, tail_v)

    lanes = lax.iota(jnp.int32, 16)
    zeros16 = jnp.zeros((16,), jnp.int32)

    def fire_loads(p, base):
        b = pl.multiple_of(base, 128)
        for a in range(_NB):
            pltpu.async_copy(
                tblt_hbm.at[:, pl.ds(pl.multiple_of(b + 128 * a, 128), 128)],
                chunk.at[p, pl.ds(64 * a, 64), :],
                sem_ld,
            )

    def wait_loads():
        for a in range(_NB):
            pltpu.make_async_copy(
                tblt_hbm.at[:, pl.ds(0, 128)],
                chunk.at[0, pl.ds(0, 64), :],
                sem_ld,
            ).wait()

    def ring_fire(g, jv):
        @pl.when(g >= 3)
        def _():
            pltpu.make_async_copy(
                g_out.at[pl.ds(0, 16), :], srcbuf.at[0], sem_sc).wait()
        pltpu.async_copy(srcbuf.at[g & 3], g_out.at[jv], sem_sc)

    def group_vecs(g):
        gi = jnp.minimum(g, _NG - 1)
        iv = sidx_v[pl.ds(gi * 16, 16)]
        jv = ord_v[pl.ds(gi * 16, 16)]
        return gi, iv, jv

    def main_phase(carry0):
        def outer_cond(c):
            gcur, umin, _, _ = c
            return (gcur < _NG) & (umin < _TAIL_START)

        def outer_body(c):
            gcur, umin, base, r = c
            p = r & 1
            wait_loads()
            chunk_p = chunk.at[p]

            def inner_cond(c2):
                _, cmin = c2
                return cmin < base + _CW

            def inner_body(c2):
                g, _ = c2
                gi, iv, jv = group_vecs(g)
                inm = (iv >= base) & (iv < base + _CW)
                colc = jnp.clip(iv - base, 0, _CW - 1)
                a64 = (colc >> 7) * 64
                within = colc & 127
                slab = srcbuf.at[gi & 3]
                for cc in range(_LATENT_DIM):
                    val = plsc.load_gather(
                        chunk_p, [a64 + cc, within], mask=inm)
                    plsc.store_scatter(
                        slab, [lanes, cc + zeros16], val, mask=inm)
                maxiv = jnp.sum(jnp.where(lanes == 15, iv, 0))
                complete = maxiv < base + _CW

                @pl.when(complete)
                def _():
                    ring_fire(gi, jv)

                ng = g + complete.astype(jnp.int32)
                _, nvec, _ = group_vecs(ng)
                thresh = jnp.where(complete, -_BIG, base + _CW)
                cmin = jnp.min(jnp.where(nvec >= thresh, nvec, _BIG))
                cmin = jnp.where(ng >= _NG, _BIG, cmin)
                return ng, cmin

            gend, uend = lax.while_loop(
                inner_cond, inner_body, (gcur, umin))

            nactual = jnp.minimum((uend >> 7) << 7, _TAIL_START - _CW)

            @pl.when(nactual != jnp.minimum(base + _CW, _TAIL_START - _CW))
            def _():
                wait_loads()
                fire_loads((r + 1) & 1, nactual)

            fire_loads(p, jnp.minimum(nactual + _CW, _TAIL_START - _CW))
            return gend, uend, nactual, r + 1

        return lax.while_loop(outer_cond, outer_body, carry0)

    def tail_phase(carry0):
        def cond(g):
            return g < _NG

        def body(g):
            gi, iv, jv = group_vecs(g)
            inm = iv >= _TAIL_START
            rr = jnp.clip(iv - _TAIL_START, 0, _TAIL_ROWS - 1)
            prow = rr >> 1
            cbase = (rr & 1) << 6
            slab = srcbuf.at[gi & 3]
            for cc in range(_LATENT_DIM):
                val = plsc.load_gather(
                    tail_v, [prow, cbase + cc], mask=inm)
                plsc.store_scatter(
                    slab, [lanes, cc + zeros16], val, mask=inm)
            ring_fire(gi, jv)
            return g + 1

        return lax.while_loop(cond, body, carry0)

    umin0 = jnp.min(sidx_v[pl.ds(0, 16)])
    b0 = jnp.minimum((umin0 >> 7) << 7, _TAIL_START - _CW)
    fire_loads(0, b0)
    fire_loads(1, jnp.minimum(b0 + _CW, _TAIL_START - _CW))

    gcur, _, _, _ = main_phase((jnp.int32(0), umin0, b0, jnp.int32(0)))
    tail_phase(gcur)

    for _ in range(2):
        wait_loads()
    for _ in range(3):
        pltpu.make_async_copy(
            g_out.at[pl.ds(0, 16), :], srcbuf.at[0], sem_sc).wait()


def kernel(uids, iids, embedding):
    ui = uids.astype(jnp.int32)
    ii = iids.astype(jnp.int32) + _ITEM_IDX_OFFSET
    allidx = jnp.concatenate([ui, ii])
    sidx, order = lax.sort_key_val(
        allidx, jnp.arange(_NT, dtype=jnp.int32))
    tail2 = embedding[_TAIL_START:, :].reshape(_TAIL_ROWS // 2, 128)
    g = _gather_sorted(sidx, order, embedding.T, tail2)
    return (g[:_BATCH, :_LATENT_DIM], g[_BATCH:, :_LATENT_DIM])


# speedup vs baseline: 2.6259x; 1.0003x over previous
import functools

import jax
import jax.numpy as jnp
from jax import lax
from jax.experimental import pallas as pl
from jax.experimental.pallas import tpu as pltpu
from jax.experimental.pallas import tpu_sc as plsc

_NUM_USERS = 100000
_NUM_ROWS = 1100000
_LATENT_DIM = 64
_BATCH = 16384
_ITEM_IDX_OFFSET = _NUM_USERS

_NC = 2
_NS = 16
_NW = _NC * _NS
_NT = 2 * _BATCH
_TPW = _NT // _NW
_NG = _TPW // 16
_CW = 768
_NB = _CW // 128
_TAIL_START = (_NUM_ROWS // 128) * 128
_TAIL_ROWS = _NUM_ROWS - _TAIL_START
_BIG = 2**31 - 1

_mesh = plsc.VectorSubcoreMesh(core_axis_name="c", subcore_axis_name="s")


@functools.partial(
    pl.kernel,
    mesh=_mesh,
    out_type=jax.ShapeDtypeStruct((_NT, 128), jnp.float32),
    scratch_types=[
        pltpu.VMEM((_TPW,), jnp.int32),
        pltpu.VMEM((_TPW,), jnp.int32),
        pltpu.VMEM((2, _LATENT_DIM, _CW), jnp.float32),
        pltpu.VMEM((4, 16, 128), jnp.float32),
        pltpu.VMEM((_TAIL_ROWS // 2, 128), jnp.float32),
        pltpu.SemaphoreType.DMA,
        pltpu.SemaphoreType.DMA,
    ],
    compiler_params=pltpu.CompilerParams(
        use_tc_tiling_on_sc=True, needs_layout_passes=False),
)
def _gather_sorted(sidx_hbm, ord_hbm, tblt_hbm, tail_hbm, g_out,
                   sidx_v, ord_v, chunk, srcbuf, tail_v, sem_ld, sem_sc):
    wid = lax.axis_index("s") * _NC + lax.axis_index("c")
    t0 = wid * _TPW

    pltpu.sync_copy(sidx_hbm.at[pl.ds(t0, _TPW)], sidx_v)
    pltpu.sync_copy(ord_hbm.at[pl.ds(t0, _TPW)], ord_v)
    pltpu.sync_copy(tail_hbm, tail_v)

    lanes = lax.iota(jnp.int32, 16)
    zeros16 = jnp.zeros((16,), jnp.int32)

    def fire_loads(p, base):
        b = pl.multiple_of(base, 128)
        pltpu.async_copy(
            tblt_hbm.at[:, pl.ds(b, _CW)], chunk.at[p], sem_ld)

    def wait_loads():
        pltpu.make_async_copy(
            tblt_hbm.at[:, pl.ds(0, _CW)], chunk.at[0], sem_ld).wait()

    def ring_fire(g, jv):
        @pl.when(g >= 3)
        def _():
            pltpu.make_async_copy(
                g_out.at[pl.ds(0, 16), :], srcbuf.at[0], sem_sc).wait()
        pltpu.async_copy(srcbuf.at[g & 3], g_out.at[jv], sem_sc)

    def group_vecs(g):
        gi = jnp.minimum(g, _NG - 1)
        iv = sidx_v[pl.ds(gi * 16, 16)]
        jv = ord_v[pl.ds(gi * 16, 16)]
        return gi, iv, jv

    def main_phase(carry0):
        def outer_cond(c):
            gcur, umin, _, _ = c
            return (gcur < _NG) & (umin < _TAIL_START)

        def outer_body(c):
            gcur, umin, base, r = c
            p = r & 1
            wait_loads()
            chunk_p = chunk.at[p]

            def inner_cond(c2):
                _, cmin = c2
                return cmin < base + _CW

            def inner_body(c2):
                g, _ = c2
                gi, iv, jv = group_vecs(g)
                inm = (iv >= base) & (iv < base + _CW)
                colc = jnp.clip(iv - base, 0, _CW - 1)
                slab = srcbuf.at[gi & 3]
                for cc in range(_LATENT_DIM):
                    val = plsc.load_gather(
                        chunk_p, [cc + zeros16, colc], mask=inm)
                    plsc.store_scatter(
                        slab, [lanes, cc + zeros16], val, mask=inm)
                maxiv = jnp.sum(jnp.where(lanes == 15, iv, 0))
                complete = maxiv < base + _CW

                @pl.when(complete)
                def _():
                    ring_fire(gi, jv)

                ng = g + complete.astype(jnp.int32)
                _, nvec, _ = group_vecs(ng)
                thresh = jnp.where(complete, -_BIG, base + _CW)
                cmin = jnp.min(jnp.where(nvec >= thresh, nvec, _BIG))
                cmin = jnp.where(ng >= _NG, _BIG, cmin)
                return ng, cmin

            gend, uend = lax.while_loop(
                inner_cond, inner_body, (gcur, umin))

            nactual = jnp.minimum((uend >> 7) << 7, _TAIL_START - _CW)

            @pl.when(nactual != jnp.minimum(base + _CW, _TAIL_START - _CW))
            def _():
                wait_loads()
                fire_loads((r + 1) & 1, nactual)

            fire_loads(p, jnp.minimum(nactual + _CW, _TAIL_START - _CW))
            return gend, uend, nactual, r + 1

        return lax.while_loop(outer_cond, outer_body, carry0)

    def tail_phase(carry0):
        def cond(g):
            return g < _NG

        def body(g):
            gi, iv, jv = group_vecs(g)
            inm = iv >= _TAIL_START
            rr = jnp.clip(iv - _TAIL_START, 0, _TAIL_ROWS - 1)
            prow = rr >> 1
            cbase = (rr & 1) << 6
            slab = srcbuf.at[gi & 3]
            for cc in range(_LATENT_DIM):
                val = plsc.load_gather(
                    tail_v, [prow, cbase + cc], mask=inm)
                plsc.store_scatter(
                    slab, [lanes, cc + zeros16], val, mask=inm)
            ring_fire(gi, jv)
            return g + 1

        return lax.while_loop(cond, body, carry0)

    umin0 = jnp.min(sidx_v[pl.ds(0, 16)])
    b0 = jnp.minimum((umin0 >> 7) << 7, _TAIL_START - _CW)
    fire_loads(0, b0)
    fire_loads(1, jnp.minimum(b0 + _CW, _TAIL_START - _CW))

    gcur, _, _, _ = main_phase((jnp.int32(0), umin0, b0, jnp.int32(0)))
    tail_phase(gcur)

    for _ in range(2):
        wait_loads()
    for _ in range(3):
        pltpu.make_async_copy(
            g_out.at[pl.ds(0, 16), :], srcbuf.at[0], sem_sc).wait()


def kernel(uids, iids, embedding):
    ui = uids.astype(jnp.int32)
    ii = iids.astype(jnp.int32) + _ITEM_IDX_OFFSET
    allidx = jnp.concatenate([ui, ii])
    sidx, order = lax.sort_key_val(
        allidx, jnp.arange(_NT, dtype=jnp.int32))
    tail2 = embedding[_TAIL_START:, :].reshape(_TAIL_ROWS // 2, 128)
    g = _gather_sorted(sidx, order, embedding.T, tail2)
    return (g[:_BATCH, :_LATENT_DIM], g[_BATCH:, :_LATENT_DIM])
